# baseline (speedup 1.0000x reference)
import functools

import jax
import jax.numpy as jnp
from jax import lax
from jax.experimental import pallas as pl
from jax.experimental.pallas import tpu as pltpu
from jax.experimental.pallas import tpu_sc as plsc

_EMBED = 64
_NC = 2
_NS = 16
_NW = _NC * _NS

_IDXROW = 128
_SUB = 4
_CH = _IDXROW * _SUB
_NBUF = 2


def _emb_body(idx_hbm, table_hbm, out_hbm, idx_v, rows_v, gsem, ssem):
    n_rows_w = idx_v.shape[0]
    n_chunks = n_rows_w // _SUB
    wid = lax.axis_index("s") * _NC + lax.axis_index("c")
    base = wid * (n_rows_w * _IDXROW)

    pltpu.sync_copy(idx_hbm.at[wid], idx_v)

    def gather_start(b, c):
        for j in range(_SUB):
            pltpu.async_copy(
                table_hbm.at[idx_v.at[c * _SUB + j]],
                rows_v.at[b, pl.ds(j * _IDXROW, _IDXROW)],
                gsem.at[b],
            )

    def gather_wait(b):
        pltpu.make_async_copy(
            table_hbm.at[pl.ds(0, _CH)], rows_v.at[b], gsem.at[b]
        ).wait()

    def store_start(b, c):
        pltpu.async_copy(
            rows_v.at[b], out_hbm.at[pl.ds(base + c * _CH, _CH)], ssem.at[b]
        )

    def store_wait(b):
        pltpu.make_async_copy(
            rows_v.at[b], out_hbm.at[pl.ds(0, _CH)], ssem.at[b]
        ).wait()

    for b in range(_NBUF):
        gather_start(b, b)

    @pl.loop(0, n_chunks - _NBUF, step=_NBUF)
    def _steady(c0):
        for b in range(_NBUF):
            c = c0 + b
            gather_wait(b)
            store_start(b, c)
            store_wait(b)
            gather_start(b, c + _NBUF)

    for j in range(_NBUF):
        c = n_chunks - _NBUF + j
        gather_wait(j)
        store_start(j, c)
    for b in range(_NBUF):
        store_wait(b)


@jax.jit
def kernel(x, word_embedding):
    batch, seq = x.shape
    vocab, embed = word_embedding.shape
    total = batch * seq
    n_rows_w = total // (_NW * _IDXROW)
    idx = x.reshape(_NW, n_rows_w, _IDXROW).astype(jnp.int32)

    mesh = plsc.VectorSubcoreMesh(core_axis_name="c", subcore_axis_name="s")
    grab = pl.kernel(
        _emb_body,
        out_type=jax.ShapeDtypeStruct((total, embed), jnp.float32),
        mesh=mesh,
        scratch_types=[
            pltpu.VMEM((n_rows_w, _IDXROW), jnp.int32),
            pltpu.VMEM((_NBUF, _CH, embed), jnp.float32),
            pltpu.SemaphoreType.DMA((_NBUF,)),
            pltpu.SemaphoreType.DMA((_NBUF,)),
        ],
        compiler_params=pltpu.CompilerParams(use_tc_tiling_on_sc=False),
    )
    out = grab(idx, word_embedding)
    return out.reshape(batch, seq, embed)
